# Initial kernel scaffold; baseline (speedup 1.0000x reference)
import jax
import jax.numpy as jnp
from jax.experimental import pallas as pl


def kernel(x_base, x_local, edge_index, enc_w1, enc_b1, gn1_g, gn1_b, enc_w2, enc_b2, gn2_g, gn2_b, mlp_w, mlp_b, g1_wl, g1_bl, g1_wr, g1_br, g1_att, g1_bias, g2_wl, g2_bl, g2_wr, g2_br, g2_att, g2_bias, tc_w1, tc_b1, tc_w2, tc_b2, attn_in_w, attn_in_b, attn_out_w, attn_out_b, ff_w1, ff_b1, ff_w2, ff_b2, ln1_g, ln1_b, ln2_g, ln2_b, fus_w, fus_b, reg_w, reg_b):
    raise NotImplementedError("write your pallas kernel here")


# speedup vs baseline: 1.1673x; 1.1673x over previous
import jax
import jax.numpy as jnp
from jax import lax
from jax.experimental import pallas as pl
from jax.experimental.pallas import tpu as pltpu

_INTERPRET = False

_N = 1000
_NP = 1024
_K = 5
_D = 64


def _topk_body(sim_ref, out_ref):
    sim = sim_ref[0, 0]
    lane = lax.broadcasted_iota(jnp.int32, (_NP, _NP), 1)
    lane8 = lax.broadcasted_iota(jnp.int32, (1, 8), 1)
    simw0 = jnp.where(lane < _N, sim, -2.0)

    def step(i, carry):
        simw, idxacc = carry
        rm = jnp.max(simw, axis=1, keepdims=True)
        jsel = jnp.min(jnp.where(simw == rm, lane, 2 * _NP),
                       axis=1, keepdims=True)
        sel = lane == jsel
        idxacc = idxacc + jsel * (lane8 == i).astype(jnp.int32)
        return jnp.where(sel, -2.0, simw), idxacc

    _, idx = lax.fori_loop(0, _K, step,
                           (simw0, jnp.zeros((_NP, 8), jnp.int32)))
    out_ref[0, 0] = idx


def _topk_call(sim_pad):
    T, B = sim_pad.shape[0], sim_pad.shape[1]
    return pl.pallas_call(
        _topk_body,
        grid=(T, B),
        in_specs=[pl.BlockSpec((1, 1, _NP, _NP), lambda t, b: (t, b, 0, 0))],
        out_specs=pl.BlockSpec((1, 1, _NP, 8), lambda t, b: (t, b, 0, 0)),
        out_shape=jax.ShapeDtypeStruct((T, B, _NP, 8), jnp.int32),
        compiler_params=pltpu.CompilerParams(
            dimension_semantics=("arbitrary", "arbitrary")),
        interpret=_INTERPRET,
    )(sim_pad)


def _ffln_body(x_ref, o_ref, w1_ref, b1_ref, w2_ref, b2_ref,
               ln1g_ref, ln1b_ref, ln2g_ref, ln2b_ref, out_ref):
    x = x_ref[0] + o_ref[0]

    def ln(v, g, b, eps=1e-5):
        m = v.mean(-1, keepdims=True)
        var = ((v - m) ** 2).mean(-1, keepdims=True)
        return (v - m) / jnp.sqrt(var + eps) * g + b

    x = ln(x, ln1g_ref[...], ln1b_ref[...])
    h = lax.dot_general(x, w1_ref[...], (((1,), (1,)), ((), ())),
                        preferred_element_type=jnp.float32) + b1_ref[...]
    h = jnp.maximum(h, 0.0)
    f = lax.dot_general(h, w2_ref[...], (((1,), (1,)), ((), ())),
                        preferred_element_type=jnp.float32) + b2_ref[...]
    out_ref[0] = ln(x + f, ln2g_ref[...], ln2b_ref[...])


def _ffln_call(x, o, p):
    R = x.shape[0]
    RT = 1000
    nt = R // RT
    full = lambda *s: pl.BlockSpec(s, lambda i: (0,) * len(s))
    return pl.pallas_call(
        _ffln_body,
        grid=(nt,),
        in_specs=[
            pl.BlockSpec((1, RT, _D), lambda i: (i, 0, 0)),
            pl.BlockSpec((1, RT, _D), lambda i: (i, 0, 0)),
            full(2048, _D), full(1, 2048), full(_D, 2048), full(1, _D),
            full(1, _D), full(1, _D), full(1, _D), full(1, _D),
        ],
        out_specs=pl.BlockSpec((1, RT, _D), lambda i: (i, 0, 0)),
        out_shape=jax.ShapeDtypeStruct((nt, RT, _D), jnp.float32),
        compiler_params=pltpu.CompilerParams(
            dimension_semantics=("arbitrary",)),
        interpret=_INTERPRET,
    )(x.reshape(nt, RT, _D), o.reshape(nt, RT, _D),
      p['ff_w1'], p['ff_b1'].reshape(1, -1), p['ff_w2'],
      p['ff_b2'].reshape(1, -1),
      p['ln1_g'].reshape(1, -1), p['ln1_b'].reshape(1, -1),
      p['ln2_g'].reshape(1, -1), p['ln2_b'].reshape(1, -1)).reshape(R, _D)


def _silu(x):
    return x * jax.nn.sigmoid(x)


def _conv2d(x, w, b, stride, pad):
    y = lax.conv_general_dilated(x, w, (stride, stride),
                                 ((pad, pad), (pad, pad)),
                                 dimension_numbers=('NCHW', 'OIHW', 'NCHW'))
    return y + b[None, :, None, None]


def _conv1d(x, w, b, pad, dil):
    y = lax.conv_general_dilated(x, w, (1,), ((pad, pad),),
                                 rhs_dilation=(dil,),
                                 dimension_numbers=('NCH', 'OIH', 'NCH'))
    return y + b[None, :, None]


def _group_norm(x, g, b, groups=2, eps=1e-5):
    B, C, H, W = x.shape
    xr = x.reshape(B, groups, C // groups, H, W)
    m = xr.mean(axis=(2, 3, 4), keepdims=True)
    v = xr.var(axis=(2, 3, 4), keepdims=True)
    xn = ((xr - m) / jnp.sqrt(v + eps)).reshape(B, C, H, W)
    return xn * g[None, :, None, None] + b[None, :, None, None]


def _gatv2(x, wl, bl, wr, br, att, bias, src, dst, n):
    xl = x @ wl.T + bl
    xr = x @ wr.T + br
    m = xl[src] + xr[dst]
    m = jnp.where(m > 0, m, 0.2 * m)
    logit = m @ att
    mx = jax.ops.segment_max(logit, dst, num_segments=n)
    mx = jax.lax.stop_gradient(jnp.where(jnp.isfinite(mx), mx, 0.0))
    e = jnp.exp(logit - mx[dst])
    s = jax.ops.segment_sum(e, dst, num_segments=n)
    alpha = e / (s[dst] + 1e-16)
    return jax.ops.segment_sum(alpha[:, None] * xl[src], dst,
                               num_segments=n) + bias


def kernel(x_base, x_local, edge_index, enc_w1, enc_b1, gn1_g, gn1_b,
           enc_w2, enc_b2, gn2_g, gn2_b, mlp_w, mlp_b,
           g1_wl, g1_bl, g1_wr, g1_br, g1_att, g1_bias,
           g2_wl, g2_bl, g2_wr, g2_br, g2_att, g2_bias,
           tc_w1, tc_b1, tc_w2, tc_b2,
           attn_in_w, attn_in_b, attn_out_w, attn_out_b,
           ff_w1, ff_b1, ff_w2, ff_b2,
           ln1_g, ln1_b, ln2_g, ln2_b,
           fus_w, fus_b, reg_w, reg_b):
    B, T, N = x_base.shape[0], x_base.shape[1], x_base.shape[2]

    z = x_local.reshape(-1, 7, 3, 3)
    z = _silu(_group_norm(_conv2d(z, enc_w1, enc_b1, 1, 1), gn1_g, gn1_b))
    z = _silu(_group_norm(_conv2d(z, enc_w2, enc_b2, 2, 1), gn2_g, gn2_b))
    local_feat = z.reshape(B, N, T, -1)
    xb2 = jnp.transpose(x_base, (0, 2, 1, 3)) @ mlp_w.T + mlp_b
    node_feat = jnp.concatenate([xb2, local_feat], axis=-1)

    sims = []
    for t in range(T):
        nft = node_feat[:, :, t, :]
        nrm = jnp.maximum(jnp.sqrt(jnp.sum(nft * nft, -1)), 1e-8)
        sims.append(jnp.einsum('bid,bjd->bij', nft, nft)
                    / (nrm[:, :, None] * nrm[:, None, :]))
    sim_pad = jnp.pad(jnp.stack(sims),
                      ((0, 0), (0, 0), (0, _NP - N), (0, _NP - N)))
    idx_all = _topk_call(sim_pad)[:, :, :N, :_K]

    static = jnp.concatenate([edge_index + b * N for b in range(B)],
                             axis=1).astype(jnp.int32)
    src_dyn = jnp.tile(jnp.repeat(jnp.arange(N, dtype=jnp.int32), _K), B)
    outs = []
    for t in range(T):
        nf_t = node_feat[:, :, t, :]
        dst_dyn = idx_all[t].reshape(-1)
        src = jnp.concatenate([static[0], src_dyn])
        dst = jnp.concatenate([static[1], dst_dyn])
        x = nf_t.reshape(-1, _D)
        h = _gatv2(x, g1_wl, g1_bl, g1_wr, g1_br, g1_att, g1_bias,
                   src, dst, B * N)
        h = jnp.maximum(h, 0)
        h = _gatv2(h, g2_wl, g2_bl, g2_wr, g2_br, g2_att, g2_bias,
                   src, dst, B * N)
        h = h.reshape(B, N, -1)
        if t > 0:
            ht = jnp.transpose(h, (0, 2, 1))
            ht = _conv1d(ht, tc_w1, tc_b1, 1, 1)
            ht = jnp.maximum(ht, 0)
            ht = _conv1d(ht, tc_w2, tc_b2, 2, 2)
            h = h + jnp.transpose(ht, (0, 2, 1))
        outs.append(h)
    to = jnp.stack(outs, axis=1)

    hd = to.shape[-1]
    x = to.reshape(-1, N, hd)
    S, Nb = x.shape[0], x.shape[1]
    H = 4
    dh = hd // H
    qkv = x @ attn_in_w.T + attn_in_b
    q, k, v = jnp.split(qkv, 3, axis=-1)
    q = q.reshape(S, Nb, H, dh)
    k = k.reshape(S, Nb, H, dh)
    v = v.reshape(S, Nb, H, dh)
    scores = jnp.einsum('snhd,tnhd->nhst', q, k) / jnp.sqrt(jnp.float32(dh))
    a = jax.nn.softmax(scores, axis=-1)
    o = jnp.einsum('nhst,tnhd->snhd', a, v).reshape(S, Nb, hd)
    o = o @ attn_out_w.T + attn_out_b
    p = {'ff_w1': ff_w1, 'ff_b1': ff_b1, 'ff_w2': ff_w2, 'ff_b2': ff_b2,
         'ln1_g': ln1_g, 'ln1_b': ln1_b, 'ln2_g': ln2_g, 'ln2_b': ln2_b}
    xs = _ffln_call(x.reshape(-1, hd), o.reshape(-1, hd), p)
    to = xs.reshape(B, T, N, hd)
    tmax = jnp.max(to, axis=1)

    preds = []
    for t in range(T):
        sp = jnp.mean(to[:, t, :, :], axis=-1, keepdims=True)
        fused = jnp.concatenate([sp, tmax], axis=-1) @ fus_w.T + fus_b
        preds.append((fused @ reg_w.T + reg_b)[..., 0])
    return jnp.stack(preds, axis=1)
